# baseline (speedup 1.0000x reference)
import functools

import jax
import jax.numpy as jnp
from jax import lax
from jax.experimental import pallas as pl
from jax.experimental.pallas import tpu as pltpu
from jax.experimental.pallas import tpu_sc as plsc

N = 10000
E = 320000
B = 64
HID = 128

NC = 2
NS = 16
CH = 88

NBUF = 4
E_PAD = 326656
EPS = E_PAD // NS
NCHUNK = EPS // CH
NB = NCHUNK // NBUF
NJ = 10008
RPS = 624
RLAST = N - 15 * RPS

NH = 10240
CPS = NH // NS

_PREC = lax.Precision.HIGHEST


@functools.cache
def _sc_kernels():
    mesh = plsc.VectorSubcoreMesh(core_axis_name="c", subcore_axis_name="s",
                                  num_cores=NC, num_subcores=NS)

    @functools.partial(
        pl.kernel,
        out_type=jax.ShapeDtypeStruct((NC, N, HID), jnp.float32),
        mesh=mesh,
        scratch_types=[
            pltpu.VMEM((NCHUNK, CH), jnp.int32),
            pltpu.VMEM((CH, HID), jnp.float32),
            pltpu.VMEM_SHARED((NJ, HID), jnp.float32),
            pltpu.SemaphoreType.DMA,
        ],
    )
    def deg_kernel(dst4, ones_hbm, out, idx_all, ones_v, acc_sh, sem):
        c = lax.axis_index("c")
        s = lax.axis_index("s")
        r0 = pl.multiple_of(s * RPS, 8)
        rl = pl.multiple_of((NS - 1) * RPS, 8)

        @pl.when(s < NS - 1)
        def _():
            pltpu.sync_copy(ones_hbm.at[pl.ds(r0, RPS)],
                            acc_sh.at[pl.ds(r0, RPS)])

        @pl.when(s == NS - 1)
        def _():
            pltpu.sync_copy(ones_hbm.at[pl.ds(rl, RLAST)],
                            acc_sh.at[pl.ds(rl, RLAST)])

        pltpu.sync_copy(ones_hbm.at[pl.ds(0, CH)], ones_v)
        pltpu.sync_copy(dst4.at[c, s], idx_all)
        plsc.subcore_barrier()

        def body(j, carry):
            for b in range(NBUF):
                i = j * NBUF + b
                pltpu.async_copy(ones_v, acc_sh.at[idx_all.at[i]], sem,
                                 add=True)
            for b in range(NBUF):
                pltpu.make_async_copy(ones_v, acc_sh.at[idx_all.at[j * NBUF]],
                                      sem).wait()
            return carry

        lax.fori_loop(0, NCHUNK // NBUF, body, 0)
        plsc.subcore_barrier()

        @pl.when(s < NS - 1)
        def _():
            pltpu.sync_copy(acc_sh.at[pl.ds(r0, RPS)],
                            out.at[c, pl.ds(r0, RPS)])

        @pl.when(s == NS - 1)
        def _():
            pltpu.sync_copy(acc_sh.at[pl.ds(rl, RLAST)],
                            out.at[c, pl.ds(rl, RLAST)])

    @functools.partial(
        pl.kernel,
        out_type=jax.ShapeDtypeStruct((NC, N, HID), jnp.float32),
        mesh=mesh,
        scratch_types=[
            pltpu.VMEM((2, NBUF, CH), jnp.int32),
            pltpu.VMEM((2, NBUF, CH), jnp.int32),
            pltpu.VMEM((NBUF, CH, HID), jnp.float32),
            pltpu.VMEM_SHARED((NJ, HID), jnp.float32),
            pltpu.SemaphoreType.DMA,
            pltpu.SemaphoreType.DMA,
            pltpu.SemaphoreType.DMA,
            pltpu.SemaphoreType.DMA,
            pltpu.SemaphoreType.DMA,
            pltpu.SemaphoreType.DMA,
            pltpu.SemaphoreType.DMA,
        ],
    )
    def scatter_kernel(h_hbm, src_hbm, dst_hbm, out, srcb, dstb, bufs,
                       acc_sh, gsem0, gsem1, gsem2, gsem3, ssem, isem0, isem1):
        c = lax.axis_index("c")
        s = lax.axis_index("s")
        gsems = [gsem0, gsem1, gsem2, gsem3]
        r0 = pl.multiple_of(s * RPS, 8)
        rl = pl.multiple_of((NS - 1) * RPS, 8)

        @pl.when(s < NS - 1)
        def _():
            pltpu.sync_copy(h_hbm.at[pl.ds(c * N + r0, RPS)],
                            acc_sh.at[pl.ds(r0, RPS)])

        @pl.when(s == NS - 1)
        def _():
            pltpu.sync_copy(h_hbm.at[pl.ds(c * N + rl, RLAST)],
                            acc_sh.at[pl.ds(rl, RLAST)])

        plsc.subcore_barrier()

        isems = [isem0, isem1]

        def load_idx_batch(j, par):
            for b in range(NBUF):
                pltpu.async_copy(src_hbm.at[c, s, j * NBUF + b],
                                 srcb.at[par, b], isems[par])
                pltpu.async_copy(dst_hbm.at[c, s, j * NBUF + b],
                                 dstb.at[par, b], isems[par])

        def drain_idx_batch(par):
            for b in range(NBUF):
                pltpu.make_async_copy(src_hbm.at[c, s, b], srcb.at[par, b],
                                      isems[par]).wait()
                pltpu.make_async_copy(dst_hbm.at[c, s, b], dstb.at[par, b],
                                      isems[par]).wait()

        load_idx_batch(0, 0)
        drain_idx_batch(0)
        load_idx_batch(1, 1)

        def batch(j, par):
            for b in range(NBUF):
                pltpu.async_copy(h_hbm.at[srcb.at[par, b]], bufs.at[b],
                                 gsems[b])
            for b in range(NBUF):
                pltpu.make_async_copy(h_hbm.at[srcb.at[par, b]], bufs.at[b],
                                      gsems[b]).wait()
                pltpu.async_copy(bufs.at[b], acc_sh.at[dstb.at[par, b]], ssem,
                                 add=True)
            for b in range(NBUF):
                pltpu.make_async_copy(bufs.at[b], acc_sh.at[dstb.at[par, b]],
                                      ssem).wait()

        def body(j2, carry):
            j0 = j2 * 2
            batch(j0, 0)
            drain_idx_batch(1)

            @pl.when(j0 + 2 < NB)
            def _():
                load_idx_batch(j0 + 2, 0)

            batch(j0 + 1, 1)

            @pl.when(j0 + 3 < NB)
            def _():
                load_idx_batch(j0 + 3, 1)

            @pl.when(j0 + 2 < NB)
            def _():
                drain_idx_batch(0)

            return carry

        lax.fori_loop(0, NB // 2, body, 0)
        plsc.subcore_barrier()

        @pl.when(s < NS - 1)
        def _():
            pltpu.sync_copy(acc_sh.at[pl.ds(r0, RPS)], out.at[c, pl.ds(r0, RPS)])

        @pl.when(s == NS - 1)
        def _():
            pltpu.sync_copy(acc_sh.at[pl.ds(rl, RLAST)],
                            out.at[c, pl.ds(rl, RLAST)])

    return deg_kernel, scatter_kernel


BLK = 2000


def _prelu(x, a):
    return jnp.where(x >= 0.0, x, a * x)


def _enc_body(x_ref, wenc_ref, w1_ref, deg_ref, aact_ref, h1_ref):
    x = x_ref[0]
    enc = jnp.dot(x, wenc_ref[...], preferred_element_type=jnp.float32,
                  precision=_PREC)
    enc = _prelu(enc, aact_ref[0])
    dinv = lax.rsqrt(deg_ref[0, :, 0:1])
    h = jnp.dot(enc, w1_ref[0], preferred_element_type=jnp.float32,
                precision=_PREC)
    h1_ref[0] = h * dinv


def _enc_call(x, wenc, w1s, deg3, aact):
    return pl.pallas_call(
        _enc_body,
        grid=(2, N // BLK),
        in_specs=[
            pl.BlockSpec((1, BLK, HID), lambda g, i: (g, i, 0)),
            pl.BlockSpec((HID, HID), lambda g, i: (0, 0)),
            pl.BlockSpec((1, HID, HID), lambda g, i: (g, 0, 0)),
            pl.BlockSpec((1, BLK, HID), lambda g, i: (g, i, 0)),
            pl.BlockSpec(memory_space=pltpu.SMEM),
        ],
        out_specs=pl.BlockSpec((1, BLK, HID), lambda g, i: (g, i, 0)),
        out_shape=jax.ShapeDtypeStruct((2, N, HID), jnp.float32),
    )(x, wenc, w1s, deg3, aact)


def _mid_body(acc_ref, deg_ref, b_ref, av_ref, w2_ref, z1_ref, h2_ref):
    dinv = lax.rsqrt(deg_ref[0, :, 0:1])
    z = _prelu(acc_ref[0] * dinv + b_ref[0], av_ref[0])
    z1_ref[0] = z
    h2_ref[0] = jnp.dot(z, w2_ref[0], preferred_element_type=jnp.float32,
                        precision=_PREC) * dinv


def _mid_call(acc1, deg3, b1s, avs, w2s):
    return pl.pallas_call(
        _mid_body,
        grid=(2, N // BLK),
        in_specs=[
            pl.BlockSpec((1, BLK, HID), lambda g, i: (g, i, 0)),
            pl.BlockSpec((1, BLK, HID), lambda g, i: (g, i, 0)),
            pl.BlockSpec((1, 1, HID), lambda g, i: (g, 0, 0)),
            pl.BlockSpec((1, 1, HID), lambda g, i: (g, 0, 0)),
            pl.BlockSpec((1, HID, HID), lambda g, i: (g, 0, 0)),
        ],
        out_specs=[
            pl.BlockSpec((1, BLK, HID), lambda g, i: (g, i, 0)),
            pl.BlockSpec((1, BLK, HID), lambda g, i: (g, i, 0)),
        ],
        out_shape=[
            jax.ShapeDtypeStruct((2, N, HID), jnp.float32),
            jax.ShapeDtypeStruct((2, N, HID), jnp.float32),
        ],
    )(acc1, deg3, b1s, avs, w2s)


def _pool_body(acc_ref, deg_ref, b_ref, av_ref, z1_ref, batch_ref,
               z2_ref, g_ref):
    dinv = lax.rsqrt(deg_ref[0, :, 0:1])
    z2 = _prelu(acc_ref[0] * dinv + b_ref[0], av_ref[0])
    z2_ref[0] = z2
    bat = batch_ref[0]
    seg = lax.broadcasted_iota(jnp.int32, (B, N), 0)
    p = (bat == seg).astype(jnp.float32)
    cnt = jnp.maximum(jnp.sum(p, axis=1, keepdims=True), 1.0)
    s1 = jnp.dot(p, z1_ref[0], preferred_element_type=jnp.float32,
                 precision=_PREC)
    s2 = jnp.dot(p, z2, preferred_element_type=jnp.float32, precision=_PREC)
    g_ref[0] = jnp.concatenate([s1 / cnt, s2 / cnt], axis=1)


def _pool_call(acc2, deg3, b2s, avs, z1, batch3):
    return pl.pallas_call(
        _pool_body,
        grid=(2,),
        in_specs=[
            pl.BlockSpec((1, N, HID), lambda g: (g, 0, 0)),
            pl.BlockSpec((1, N, HID), lambda g: (g, 0, 0)),
            pl.BlockSpec((1, 1, HID), lambda g: (g, 0, 0)),
            pl.BlockSpec((1, 1, HID), lambda g: (g, 0, 0)),
            pl.BlockSpec((1, N, HID), lambda g: (g, 0, 0)),
            pl.BlockSpec((1, 1, N), lambda g: (g, 0, 0)),
        ],
        out_specs=[
            pl.BlockSpec((1, N, HID), lambda g: (g, 0, 0)),
            pl.BlockSpec((1, B, 2 * HID), lambda g: (g, 0, 0)),
        ],
        out_shape=[
            jax.ShapeDtypeStruct((2, N, HID), jnp.float32),
            jax.ShapeDtypeStruct((2, B, 2 * HID), jnp.float32),
        ],
    )(acc2, deg3, b2s, avs, z1, batch3)


def _mlp1_body(x_ref, w_ref, b_ref, a_ref, out_ref):
    x = x_ref[...]
    h = jnp.dot(x, w_ref[0], preferred_element_type=jnp.float32,
                precision=_PREC) + b_ref[0:1, :]
    h = _prelu(h, a_ref[0])
    h = jnp.dot(h, w_ref[1], preferred_element_type=jnp.float32,
                precision=_PREC) + b_ref[1:2, :]
    h = _prelu(h, a_ref[1])
    h = jnp.dot(h, w_ref[2], preferred_element_type=jnp.float32,
                precision=_PREC) + b_ref[2:3, :]
    h = _prelu(h, a_ref[2])
    skip = jnp.dot(x, w_ref[3], preferred_element_type=jnp.float32,
                   precision=_PREC) + b_ref[3:4, :]
    out_ref[...] = h + skip


def _mlp1_call(x, ws, bs, avals):
    return pl.pallas_call(
        _mlp1_body,
        grid=(2 * N // BLK,),
        in_specs=[
            pl.BlockSpec((BLK, HID), lambda i: (i, 0)),
            pl.BlockSpec((4, HID, HID), lambda i: (0, 0, 0)),
            pl.BlockSpec((4, HID), lambda i: (0, 0)),
            pl.BlockSpec(memory_space=pltpu.SMEM),
        ],
        out_specs=pl.BlockSpec((BLK, HID), lambda i: (i, 0)),
        out_shape=jax.ShapeDtypeStruct((2 * N, HID), jnp.float32),
    )(x, ws, bs, avals)


def _mlp2_body(x_ref, w1_ref, w23_ref, ws_ref, b_ref, a_ref, out_ref):
    x = x_ref[...]
    h = jnp.dot(x, w1_ref[...], preferred_element_type=jnp.float32,
                precision=_PREC) + b_ref[0:1, :]
    h = _prelu(h, a_ref[0])
    h = jnp.dot(h, w23_ref[0], preferred_element_type=jnp.float32,
                precision=_PREC) + b_ref[1:2, :]
    h = _prelu(h, a_ref[1])
    h = jnp.dot(h, w23_ref[1], preferred_element_type=jnp.float32,
                precision=_PREC) + b_ref[2:3, :]
    h = _prelu(h, a_ref[2])
    skip = jnp.dot(x, ws_ref[...], preferred_element_type=jnp.float32,
                   precision=_PREC) + b_ref[3:4, :]
    out_ref[...] = h + skip


def _mlp2_call(x, w1, w23, ws, bs, avals):
    return pl.pallas_call(
        _mlp2_body,
        in_specs=[
            pl.BlockSpec((2 * B, 2 * HID), lambda: (0, 0)),
            pl.BlockSpec((2 * HID, HID), lambda: (0, 0)),
            pl.BlockSpec((2, HID, HID), lambda: (0, 0, 0)),
            pl.BlockSpec((2 * HID, HID), lambda: (0, 0)),
            pl.BlockSpec((4, HID), lambda: (0, 0)),
            pl.BlockSpec(memory_space=pltpu.SMEM),
        ],
        out_specs=pl.BlockSpec((2 * B, HID), lambda: (0, 0)),
        out_shape=jax.ShapeDtypeStruct((2 * B, HID), jnp.float32),
    )(x, w1, w23, ws, bs, avals)


def kernel(local_x, global_x, local_edge_index, global_edge_index,
           local_batch, global_batch, params):
    p = params
    lei = local_edge_index.astype(jnp.int32)
    gei = global_edge_index.astype(jnp.int32)
    pad = E_PAD - E
    zpad = jnp.zeros((pad,), jnp.int32)
    npad = jnp.full((pad,), N, jnp.int32)
    src2 = jnp.stack([
        jnp.concatenate([lei[0], zpad]),
        jnp.concatenate([gei[0], zpad]) + N,
    ])
    dst2 = jnp.stack([
        jnp.concatenate([lei[1], npad]),
        jnp.concatenate([gei[1], npad]),
    ])
    src4 = src2.reshape(2, NS, NCHUNK, CH)
    dst4 = dst2.reshape(2, NS, NCHUNK, CH)

    x = jnp.stack([local_x, global_x])
    batch3 = jnp.stack([local_batch.astype(jnp.int32),
                        global_batch.astype(jnp.int32)]).reshape(2, 1, N)

    w1s = jnp.stack([p['gnn1']['layers'][0][0], p['gnn2']['layers'][0][0]])
    b1s = jnp.stack([p['gnn1']['layers'][0][1],
                     p['gnn2']['layers'][0][1]]).reshape(2, 1, HID)
    w2s = jnp.stack([p['gnn1']['layers'][1][0], p['gnn2']['layers'][1][0]])
    b2s = jnp.stack([p['gnn1']['layers'][1][1],
                     p['gnn2']['layers'][1][1]]).reshape(2, 1, HID)
    avs = jnp.stack([p['gnn1']['a'], p['gnn2']['a']]).reshape(2, 1, HID)
    aact = jnp.reshape(p['a_act'], (1,))

    deg_kernel, scatter_kernel = _sc_kernels()
    deg3 = deg_kernel(dst4, jnp.ones((NJ, HID), jnp.float32))

    h1 = _enc_call(x, p['W_enc'], w1s, deg3, aact)
    acc1 = scatter_kernel(h1.reshape(2 * N, HID), src4, dst4)
    z1, h2 = _mid_call(acc1, deg3, b1s, avs, w2s)
    acc2 = scatter_kernel(h2.reshape(2 * N, HID), src4, dst4)
    z2, g = _pool_call(acc2, deg3, b2s, avs, z1, batch3)

    m1 = p['mlp1']
    zout = _mlp1_call(
        z2.reshape(2 * N, HID),
        jnp.stack([m1['W1'], m1['W2'], m1['W3'], m1['Ws']]),
        jnp.stack([m1['b1'], m1['b2'], m1['b3'], m1['bs']]),
        jnp.stack([m1['a1'], m1['a2'], m1['a3']]),
    )
    m2 = p['mlp2']
    gout = _mlp2_call(
        g.reshape(2 * B, 2 * HID),
        m2['W1'],
        jnp.stack([m2['W2'], m2['W3']]),
        m2['Ws'],
        jnp.stack([m2['b1'], m2['b2'], m2['b3'], m2['bs']]),
        jnp.stack([m2['a1'], m2['a2'], m2['a3']]),
    )

    lz = zout[:N]
    gz = zout[N:]
    lg = gout[:B]
    gg = gout[B:]
    return (lz, lg, gz, gg)


# speedup vs baseline: 1.5326x; 1.5326x over previous
import functools

import jax
import jax.numpy as jnp
from jax import lax
from jax.experimental import pallas as pl
from jax.experimental.pallas import tpu as pltpu
from jax.experimental.pallas import tpu_sc as plsc

N = 10000
E = 320000
B = 64
HID = 128

NC = 2
NS = 16
CH = 120

NBUF = 3
E_PAD = 322560
EPS = E_PAD // NS
NCHUNK = EPS // CH
NB = NCHUNK // NBUF
NJ = 10008
RPS = 624
RLAST = N - 15 * RPS

NH = 10240
CPS = NH // NS

_PREC = None


@functools.cache
def _sc_kernels():
    mesh = plsc.VectorSubcoreMesh(core_axis_name="c", subcore_axis_name="s",
                                  num_cores=NC, num_subcores=NS)

    @functools.partial(
        pl.kernel,
        out_type=jax.ShapeDtypeStruct((NC, N, HID), jnp.float32),
        mesh=mesh,
        scratch_types=[
            pltpu.VMEM((NCHUNK, CH), jnp.int32),
            pltpu.VMEM((CH, HID), jnp.float32),
            pltpu.VMEM_SHARED((NJ, HID), jnp.float32),
            pltpu.SemaphoreType.DMA,
        ],
    )
    def deg_kernel(dst4, ones_hbm, out, idx_all, ones_v, acc_sh, sem):
        c = lax.axis_index("c")
        s = lax.axis_index("s")
        r0 = pl.multiple_of(s * RPS, 8)
        rl = pl.multiple_of((NS - 1) * RPS, 8)

        @pl.when(s < NS - 1)
        def _():
            pltpu.sync_copy(ones_hbm.at[pl.ds(r0, RPS)],
                            acc_sh.at[pl.ds(r0, RPS)])

        @pl.when(s == NS - 1)
        def _():
            pltpu.sync_copy(ones_hbm.at[pl.ds(rl, RLAST)],
                            acc_sh.at[pl.ds(rl, RLAST)])

        pltpu.sync_copy(ones_hbm.at[pl.ds(0, CH)], ones_v)
        pltpu.sync_copy(dst4.at[c, s], idx_all)
        plsc.subcore_barrier()

        def body(j, carry):
            for b in range(NBUF):
                i = j * NBUF + b
                pltpu.async_copy(ones_v, acc_sh.at[idx_all.at[i]], sem,
                                 add=True)
            for b in range(NBUF):
                pltpu.make_async_copy(ones_v, acc_sh.at[idx_all.at[j * NBUF]],
                                      sem).wait()
            return carry

        lax.fori_loop(0, NCHUNK // NBUF, body, 0)
        plsc.subcore_barrier()

        @pl.when(s < NS - 1)
        def _():
            pltpu.sync_copy(acc_sh.at[pl.ds(r0, RPS)],
                            out.at[c, pl.ds(r0, RPS)])

        @pl.when(s == NS - 1)
        def _():
            pltpu.sync_copy(acc_sh.at[pl.ds(rl, RLAST)],
                            out.at[c, pl.ds(rl, RLAST)])

    @functools.partial(
        pl.kernel,
        out_type=jax.ShapeDtypeStruct((NC, N, HID), jnp.float32),
        mesh=mesh,
        scratch_types=[
            pltpu.VMEM((2, NBUF, CH), jnp.int32),
            pltpu.VMEM((2, NBUF, CH), jnp.int32),
            pltpu.VMEM((NBUF, CH, HID), jnp.float32),
            pltpu.VMEM_SHARED((NJ, HID), jnp.float32),
            pltpu.SemaphoreType.DMA,
            pltpu.SemaphoreType.DMA,
            pltpu.SemaphoreType.DMA,
            pltpu.SemaphoreType.DMA,
            pltpu.SemaphoreType.DMA,
            pltpu.SemaphoreType.DMA,
        ],
    )
    def scatter_kernel(h_hbm, src_hbm, dst_hbm, out, srcb, dstb, bufs,
                       acc_sh, gsem0, gsem1, gsem2, ssem, isem0, isem1):
        c = lax.axis_index("c")
        s = lax.axis_index("s")
        gsems = [gsem0, gsem1, gsem2]
        r0 = pl.multiple_of(s * RPS, 8)
        rl = pl.multiple_of((NS - 1) * RPS, 8)

        @pl.when(s < NS - 1)
        def _():
            pltpu.sync_copy(h_hbm.at[pl.ds(c * N + r0, RPS)],
                            acc_sh.at[pl.ds(r0, RPS)])

        @pl.when(s == NS - 1)
        def _():
            pltpu.sync_copy(h_hbm.at[pl.ds(c * N + rl, RLAST)],
                            acc_sh.at[pl.ds(rl, RLAST)])

        plsc.subcore_barrier()

        isems = [isem0, isem1]

        def load_idx_batch(j, par):
            for b in range(NBUF):
                pltpu.async_copy(src_hbm.at[c, s, j * NBUF + b],
                                 srcb.at[par, b], isems[par])
                pltpu.async_copy(dst_hbm.at[c, s, j * NBUF + b],
                                 dstb.at[par, b], isems[par])

        def drain_idx_batch(par):
            for b in range(NBUF):
                pltpu.make_async_copy(src_hbm.at[c, s, b], srcb.at[par, b],
                                      isems[par]).wait()
                pltpu.make_async_copy(dst_hbm.at[c, s, b], dstb.at[par, b],
                                      isems[par]).wait()

        load_idx_batch(0, 0)
        drain_idx_batch(0)
        load_idx_batch(1, 1)

        def batch(j, par):
            for b in range(NBUF):
                pltpu.async_copy(h_hbm.at[srcb.at[par, b]], bufs.at[b],
                                 gsems[b])
            for b in range(NBUF):
                pltpu.make_async_copy(h_hbm.at[srcb.at[par, b]], bufs.at[b],
                                      gsems[b]).wait()
                pltpu.async_copy(bufs.at[b], acc_sh.at[dstb.at[par, b]], ssem,
                                 add=True)
            for b in range(NBUF):
                pltpu.make_async_copy(bufs.at[b], acc_sh.at[dstb.at[par, b]],
                                      ssem).wait()

        def body(j2, carry):
            j0 = j2 * 2
            batch(j0, 0)
            drain_idx_batch(1)

            @pl.when(j0 + 2 < NB)
            def _():
                load_idx_batch(j0 + 2, 0)

            batch(j0 + 1, 1)

            @pl.when(j0 + 3 < NB)
            def _():
                load_idx_batch(j0 + 3, 1)

            @pl.when(j0 + 2 < NB)
            def _():
                drain_idx_batch(0)

            return carry

        lax.fori_loop(0, NB // 2, body, 0)
        plsc.subcore_barrier()

        @pl.when(s < NS - 1)
        def _():
            pltpu.sync_copy(acc_sh.at[pl.ds(r0, RPS)], out.at[c, pl.ds(r0, RPS)])

        @pl.when(s == NS - 1)
        def _():
            pltpu.sync_copy(acc_sh.at[pl.ds(rl, RLAST)],
                            out.at[c, pl.ds(rl, RLAST)])

    return deg_kernel, scatter_kernel


BLK = 2000


def _prelu(x, a):
    return jnp.where(x >= 0.0, x, a * x)


def _enc_body(x_ref, wenc_ref, w1_ref, deg_ref, aact_ref, h1_ref):
    x = x_ref[0]
    enc = jnp.dot(x, wenc_ref[...], preferred_element_type=jnp.float32,
                  precision=_PREC)
    enc = _prelu(enc, aact_ref[0])
    dinv = lax.rsqrt(deg_ref[0, :, 0:1])
    h = jnp.dot(enc, w1_ref[0], preferred_element_type=jnp.float32,
                precision=_PREC)
    h1_ref[0] = h * dinv


def _enc_call(x, wenc, w1s, deg3, aact):
    return pl.pallas_call(
        _enc_body,
        grid=(2, N // BLK),
        in_specs=[
            pl.BlockSpec((1, BLK, HID), lambda g, i: (g, i, 0)),
            pl.BlockSpec((HID, HID), lambda g, i: (0, 0)),
            pl.BlockSpec((1, HID, HID), lambda g, i: (g, 0, 0)),
            pl.BlockSpec((1, BLK, HID), lambda g, i: (g, i, 0)),
            pl.BlockSpec(memory_space=pltpu.SMEM),
        ],
        out_specs=pl.BlockSpec((1, BLK, HID), lambda g, i: (g, i, 0)),
        out_shape=jax.ShapeDtypeStruct((2, N, HID), jnp.float32),
    )(x, wenc, w1s, deg3, aact)


def _mid_body(acc_ref, deg_ref, b_ref, av_ref, w2_ref, z1_ref, h2_ref):
    dinv = lax.rsqrt(deg_ref[0, :, 0:1])
    z = _prelu(acc_ref[0] * dinv + b_ref[0], av_ref[0])
    z1_ref[0] = z
    h2_ref[0] = jnp.dot(z, w2_ref[0], preferred_element_type=jnp.float32,
                        precision=_PREC) * dinv


def _mid_call(acc1, deg3, b1s, avs, w2s):
    return pl.pallas_call(
        _mid_body,
        grid=(2, N // BLK),
        in_specs=[
            pl.BlockSpec((1, BLK, HID), lambda g, i: (g, i, 0)),
            pl.BlockSpec((1, BLK, HID), lambda g, i: (g, i, 0)),
            pl.BlockSpec((1, 1, HID), lambda g, i: (g, 0, 0)),
            pl.BlockSpec((1, 1, HID), lambda g, i: (g, 0, 0)),
            pl.BlockSpec((1, HID, HID), lambda g, i: (g, 0, 0)),
        ],
        out_specs=[
            pl.BlockSpec((1, BLK, HID), lambda g, i: (g, i, 0)),
            pl.BlockSpec((1, BLK, HID), lambda g, i: (g, i, 0)),
        ],
        out_shape=[
            jax.ShapeDtypeStruct((2, N, HID), jnp.float32),
            jax.ShapeDtypeStruct((2, N, HID), jnp.float32),
        ],
    )(acc1, deg3, b1s, avs, w2s)


def _pool_body(acc_ref, deg_ref, b_ref, av_ref, z1_ref, batch_ref,
               z2_ref, g_ref):
    dinv = lax.rsqrt(deg_ref[0, :, 0:1])
    z2 = _prelu(acc_ref[0] * dinv + b_ref[0], av_ref[0])
    z2_ref[0] = z2
    bat = batch_ref[0]
    seg = lax.broadcasted_iota(jnp.int32, (B, N), 0)
    p = (bat == seg).astype(jnp.float32)
    cnt = jnp.maximum(jnp.sum(p, axis=1, keepdims=True), 1.0)
    s1 = jnp.dot(p, z1_ref[0], preferred_element_type=jnp.float32,
                 precision=_PREC)
    s2 = jnp.dot(p, z2, preferred_element_type=jnp.float32, precision=_PREC)
    g_ref[0] = jnp.concatenate([s1 / cnt, s2 / cnt], axis=1)


def _pool_call(acc2, deg3, b2s, avs, z1, batch3):
    return pl.pallas_call(
        _pool_body,
        grid=(2,),
        in_specs=[
            pl.BlockSpec((1, N, HID), lambda g: (g, 0, 0)),
            pl.BlockSpec((1, N, HID), lambda g: (g, 0, 0)),
            pl.BlockSpec((1, 1, HID), lambda g: (g, 0, 0)),
            pl.BlockSpec((1, 1, HID), lambda g: (g, 0, 0)),
            pl.BlockSpec((1, N, HID), lambda g: (g, 0, 0)),
            pl.BlockSpec((1, 1, N), lambda g: (g, 0, 0)),
        ],
        out_specs=[
            pl.BlockSpec((1, N, HID), lambda g: (g, 0, 0)),
            pl.BlockSpec((1, B, 2 * HID), lambda g: (g, 0, 0)),
        ],
        out_shape=[
            jax.ShapeDtypeStruct((2, N, HID), jnp.float32),
            jax.ShapeDtypeStruct((2, B, 2 * HID), jnp.float32),
        ],
    )(acc2, deg3, b2s, avs, z1, batch3)


def _mlp1_body(x_ref, w_ref, b_ref, a_ref, out_ref):
    x = x_ref[...]
    h = jnp.dot(x, w_ref[0], preferred_element_type=jnp.float32,
                precision=_PREC) + b_ref[0:1, :]
    h = _prelu(h, a_ref[0])
    h = jnp.dot(h, w_ref[1], preferred_element_type=jnp.float32,
                precision=_PREC) + b_ref[1:2, :]
    h = _prelu(h, a_ref[1])
    h = jnp.dot(h, w_ref[2], preferred_element_type=jnp.float32,
                precision=_PREC) + b_ref[2:3, :]
    h = _prelu(h, a_ref[2])
    skip = jnp.dot(x, w_ref[3], preferred_element_type=jnp.float32,
                   precision=_PREC) + b_ref[3:4, :]
    out_ref[...] = h + skip


def _mlp1_call(x, ws, bs, avals):
    return pl.pallas_call(
        _mlp1_body,
        grid=(2 * N // BLK,),
        in_specs=[
            pl.BlockSpec((BLK, HID), lambda i: (i, 0)),
            pl.BlockSpec((4, HID, HID), lambda i: (0, 0, 0)),
            pl.BlockSpec((4, HID), lambda i: (0, 0)),
            pl.BlockSpec(memory_space=pltpu.SMEM),
        ],
        out_specs=pl.BlockSpec((BLK, HID), lambda i: (i, 0)),
        out_shape=jax.ShapeDtypeStruct((2 * N, HID), jnp.float32),
    )(x, ws, bs, avals)


def _mlp2_body(x_ref, w1_ref, w23_ref, ws_ref, b_ref, a_ref, out_ref):
    x = x_ref[...]
    h = jnp.dot(x, w1_ref[...], preferred_element_type=jnp.float32,
                precision=_PREC) + b_ref[0:1, :]
    h = _prelu(h, a_ref[0])
    h = jnp.dot(h, w23_ref[0], preferred_element_type=jnp.float32,
                precision=_PREC) + b_ref[1:2, :]
    h = _prelu(h, a_ref[1])
    h = jnp.dot(h, w23_ref[1], preferred_element_type=jnp.float32,
                precision=_PREC) + b_ref[2:3, :]
    h = _prelu(h, a_ref[2])
    skip = jnp.dot(x, ws_ref[...], preferred_element_type=jnp.float32,
                   precision=_PREC) + b_ref[3:4, :]
    out_ref[...] = h + skip


def _mlp2_call(x, w1, w23, ws, bs, avals):
    return pl.pallas_call(
        _mlp2_body,
        in_specs=[
            pl.BlockSpec((2 * B, 2 * HID), lambda: (0, 0)),
            pl.BlockSpec((2 * HID, HID), lambda: (0, 0)),
            pl.BlockSpec((2, HID, HID), lambda: (0, 0, 0)),
            pl.BlockSpec((2 * HID, HID), lambda: (0, 0)),
            pl.BlockSpec((4, HID), lambda: (0, 0)),
            pl.BlockSpec(memory_space=pltpu.SMEM),
        ],
        out_specs=pl.BlockSpec((2 * B, HID), lambda: (0, 0)),
        out_shape=jax.ShapeDtypeStruct((2 * B, HID), jnp.float32),
    )(x, w1, w23, ws, bs, avals)


def kernel(local_x, global_x, local_edge_index, global_edge_index,
           local_batch, global_batch, params):
    p = params
    lei = local_edge_index.astype(jnp.int32)
    gei = global_edge_index.astype(jnp.int32)
    pad = E_PAD - E
    zpad = jnp.zeros((pad,), jnp.int32)
    npad = jnp.full((pad,), N, jnp.int32)
    src2 = jnp.stack([
        jnp.concatenate([lei[0], zpad]),
        jnp.concatenate([gei[0], zpad]) + N,
    ])
    dst2 = jnp.stack([
        jnp.concatenate([lei[1], npad]),
        jnp.concatenate([gei[1], npad]),
    ])
    src4 = src2.reshape(2, NS, NCHUNK, CH)
    dst4 = dst2.reshape(2, NS, NCHUNK, CH)

    x = jnp.stack([local_x, global_x])
    batch3 = jnp.stack([local_batch.astype(jnp.int32),
                        global_batch.astype(jnp.int32)]).reshape(2, 1, N)

    w1s = jnp.stack([p['gnn1']['layers'][0][0], p['gnn2']['layers'][0][0]])
    b1s = jnp.stack([p['gnn1']['layers'][0][1],
                     p['gnn2']['layers'][0][1]]).reshape(2, 1, HID)
    w2s = jnp.stack([p['gnn1']['layers'][1][0], p['gnn2']['layers'][1][0]])
    b2s = jnp.stack([p['gnn1']['layers'][1][1],
                     p['gnn2']['layers'][1][1]]).reshape(2, 1, HID)
    avs = jnp.stack([p['gnn1']['a'], p['gnn2']['a']]).reshape(2, 1, HID)
    aact = jnp.reshape(p['a_act'], (1,))

    deg_kernel, scatter_kernel = _sc_kernels()
    deg3 = deg_kernel(dst4, jnp.ones((NJ, HID), jnp.float32))

    h1 = _enc_call(x, p['W_enc'], w1s, deg3, aact)
    acc1 = scatter_kernel(h1.reshape(2 * N, HID), src4, dst4)
    z1, h2 = _mid_call(acc1, deg3, b1s, avs, w2s)
    acc2 = scatter_kernel(h2.reshape(2 * N, HID), src4, dst4)
    z2, g = _pool_call(acc2, deg3, b2s, avs, z1, batch3)

    m1 = p['mlp1']
    zout = _mlp1_call(
        z2.reshape(2 * N, HID),
        jnp.stack([m1['W1'], m1['W2'], m1['W3'], m1['Ws']]),
        jnp.stack([m1['b1'], m1['b2'], m1['b3'], m1['bs']]),
        jnp.stack([m1['a1'], m1['a2'], m1['a3']]),
    )
    m2 = p['mlp2']
    gout = _mlp2_call(
        g.reshape(2 * B, 2 * HID),
        m2['W1'],
        jnp.stack([m2['W2'], m2['W3']]),
        m2['Ws'],
        jnp.stack([m2['b1'], m2['b2'], m2['b3'], m2['bs']]),
        jnp.stack([m2['a1'], m2['a2'], m2['a3']]),
    )

    lz = zout[:N]
    gz = zout[N:]
    lg = gout[:B]
    gg = gout[B:]
    return (lz, lg, gz, gg)
